# Initial kernel scaffold; baseline (speedup 1.0000x reference)
import jax
import jax.numpy as jnp
from jax.experimental import pallas as pl


def kernel(entity_emb_matrix, relation_emb_matrix, items, memories_h, memories_r, memories_t):
    raise NotImplementedError("write your pallas kernel here")


# speedup vs baseline: 1.4162x; 1.4162x over previous
import functools

import jax
import jax.numpy as jnp
from jax import lax
from jax.experimental import pallas as pl
from jax.experimental.pallas import tpu as pltpu
from jax.experimental.pallas import tpu_sc as plsc

_DIM = 32
_NHOP = 2
_CHUNK = 1280


@functools.lru_cache(maxsize=None)
def _build(n_flat, n_items, dim):
    info = plsc.get_sparse_core_info()
    nw = info.num_cores * info.num_subcores
    per_w = n_flat // nw
    n_chunks = per_w // _CHUNK
    items_per_w = n_items // nw
    assert per_w * nw == n_flat and n_chunks * _CHUNK == per_w
    assert n_chunks % 2 == 0 and items_per_w * nw == n_items

    mesh = plsc.VectorSubcoreMesh(core_axis_name="c", subcore_axis_name="s")
    out3 = jax.ShapeDtypeStruct((_NHOP, n_flat, dim), jnp.float32)

    @functools.partial(
        pl.kernel,
        mesh=mesh,
        compiler_params=pltpu.CompilerParams(use_tc_tiling_on_sc=False),
        out_type=[
            jax.ShapeDtypeStruct((n_items, dim), jnp.float32),
            out3, out3, out3,
        ],
        scratch_types=[
            pltpu.VMEM((_CHUNK,), jnp.int32),
            pltpu.VMEM((_CHUNK,), jnp.int32),
            pltpu.VMEM((_CHUNK, dim), jnp.float32),
            pltpu.VMEM((_CHUNK, dim), jnp.float32),
            pltpu.VMEM((items_per_w,), jnp.int32),
            pltpu.VMEM((items_per_w, dim), jnp.float32),
            pltpu.SemaphoreType.DMA,
            pltpu.SemaphoreType.DMA,
            pltpu.SemaphoreType.DMA,
            pltpu.SemaphoreType.DMA,
            pltpu.SemaphoreType.DMA,
            pltpu.SemaphoreType.DMA,
        ],
    )
    def k(ent_hbm, rel_hbm, items_hbm, h_hbm, r_hbm, t_hbm,
          items_out, h_out, r_out, t_out,
          idx0, idx1, rows0, rows1, iidx_v, irows_v,
          sidx0, sidx1, sg0, sg1, sw0, sw1):
        wid = lax.axis_index("s") * info.num_cores + lax.axis_index("c")
        base = wid * per_w

        ibase = pl.multiple_of(wid * items_per_w, 8)
        pltpu.sync_copy(items_hbm.at[pl.ds(ibase, items_per_w)], iidx_v)
        pltpu.async_copy(ent_hbm.at[iidx_v], irows_v, sg0).wait()
        pltpu.sync_copy(irows_v, items_out.at[pl.ds(ibase, items_per_w)])

        def run_array(src_hbm, table, out):
            def src_slice(i):
                return src_hbm.at[pl.ds(pl.multiple_of(base + i * _CHUNK, 8),
                                        _CHUNK)]

            def out_slice(hop, i):
                return out.at[hop, pl.ds(pl.multiple_of(base + i * _CHUNK, 8),
                                         _CHUNK)]

            def start_idx(i, ib, sem):
                pltpu.async_copy(src_slice(i), ib, sem)

            def wait_idx(i, ib, sem):
                pltpu.make_async_copy(src_slice(i), ib, sem).wait()

            def start_gather(ib, rb, sem):
                pltpu.async_copy(table.at[ib], rb, sem)

            def wait_gather(ib, rb, sem):
                pltpu.make_async_copy(table.at[ib], rb, sem).wait()

            def start_writes(i, rb, sem):
                pltpu.async_copy(rb, out_slice(0, i), sem)
                pltpu.async_copy(rb, out_slice(1, i), sem)

            def wait_writes(i, rb, sem):
                pltpu.make_async_copy(rb, out_slice(0, i), sem).wait()
                pltpu.make_async_copy(rb, out_slice(1, i), sem).wait()

            start_idx(0, idx0, sidx0)
            start_idx(1, idx1, sidx1)
            wait_idx(0, idx0, sidx0)
            start_gather(idx0, rows0, sg0)

            def body(j, _):
                i0 = j * 2
                i1 = i0 + 1
                wait_gather(idx0, rows0, sg0)
                start_writes(i0, rows0, sw0)
                @pl.when(j < n_chunks // 2 - 1)
                def _():
                    start_idx(i0 + 2, idx0, sidx0)
                @pl.when(j > 0)
                def _():
                    wait_writes(i1 - 2, rows1, sw1)
                wait_idx(i1, idx1, sidx1)
                start_gather(idx1, rows1, sg1)
                wait_gather(idx1, rows1, sg1)
                start_writes(i1, rows1, sw1)
                @pl.when(j < n_chunks // 2 - 1)
                def _():
                    start_idx(i1 + 2, idx1, sidx1)
                    wait_writes(i0, rows0, sw0)
                    wait_idx(i0 + 2, idx0, sidx0)
                    start_gather(idx0, rows0, sg0)
                return 0

            lax.fori_loop(0, n_chunks // 2, body, 0)
            wait_writes(n_chunks - 2, rows0, sw0)
            wait_writes(n_chunks - 1, rows1, sw1)

        run_array(h_hbm, ent_hbm, h_out)
        run_array(r_hbm, rel_hbm, r_out)
        run_array(t_hbm, ent_hbm, t_out)

    return k


def kernel(entity_emb_matrix, relation_emb_matrix, items, memories_h,
           memories_r, memories_t):
    b, m = memories_h.shape
    n_flat = b * m
    run = _build(n_flat, b, _DIM)
    items_emb, h, r, t = run(
        entity_emb_matrix,
        relation_emb_matrix,
        items.astype(jnp.int32),
        memories_h.reshape(n_flat),
        memories_r.reshape(n_flat),
        memories_t.reshape(n_flat),
    )
    shp = (_NHOP, b, m, _DIM)
    return items_emb, h.reshape(shp), r.reshape(shp), t.reshape(shp)


# speedup vs baseline: 1.4177x; 1.0011x over previous
import functools

import jax
import jax.numpy as jnp
from jax import lax
from jax.experimental import pallas as pl
from jax.experimental.pallas import tpu as pltpu
from jax.experimental.pallas import tpu_sc as plsc

_DIM = 32
_NHOP = 2
_CHUNK = 640
_NBUF = 4


@functools.lru_cache(maxsize=None)
def _build(n_flat, n_items, dim):
    info = plsc.get_sparse_core_info()
    nw = info.num_cores * info.num_subcores
    per_w = n_flat // nw
    n_chunks = per_w // _CHUNK
    n_groups = n_chunks // _NBUF
    items_per_w = n_items // nw
    assert per_w * nw == n_flat and n_chunks * _CHUNK == per_w
    assert n_groups * _NBUF == n_chunks and items_per_w * nw == n_items

    mesh = plsc.VectorSubcoreMesh(core_axis_name="c", subcore_axis_name="s")
    out3 = jax.ShapeDtypeStruct((_NHOP, n_flat, dim), jnp.float32)

    @functools.partial(
        pl.kernel,
        mesh=mesh,
        compiler_params=pltpu.CompilerParams(use_tc_tiling_on_sc=False),
        out_type=[
            jax.ShapeDtypeStruct((n_items, dim), jnp.float32),
            out3, out3, out3,
        ],
        scratch_types=(
            [pltpu.VMEM((_CHUNK,), jnp.int32) for _ in range(_NBUF)]
            + [pltpu.VMEM((_CHUNK, dim), jnp.float32) for _ in range(_NBUF)]
            + [pltpu.VMEM((items_per_w,), jnp.int32),
               pltpu.VMEM((items_per_w, dim), jnp.float32)]
            + [pltpu.SemaphoreType.DMA for _ in range(3 * _NBUF)]
        ),
    )
    def k(ent_hbm, rel_hbm, items_hbm, h_hbm, r_hbm, t_hbm,
          items_out, h_out, r_out, t_out, *scratch):
        idx = scratch[:_NBUF]
        rows = scratch[_NBUF:2 * _NBUF]
        iidx_v = scratch[2 * _NBUF]
        irows_v = scratch[2 * _NBUF + 1]
        sidx = scratch[2 * _NBUF + 2:3 * _NBUF + 2]
        sg = scratch[3 * _NBUF + 2:4 * _NBUF + 2]
        sw = scratch[4 * _NBUF + 2:5 * _NBUF + 2]

        wid = lax.axis_index("s") * info.num_cores + lax.axis_index("c")
        base = wid * per_w

        ibase = pl.multiple_of(wid * items_per_w, 8)
        pltpu.sync_copy(items_hbm.at[pl.ds(ibase, items_per_w)], iidx_v)
        pltpu.async_copy(ent_hbm.at[iidx_v], irows_v, sg[0]).wait()
        pltpu.sync_copy(irows_v, items_out.at[pl.ds(ibase, items_per_w)])

        def run_array(src_hbm, table, out):
            def src_slice(i):
                return src_hbm.at[pl.ds(pl.multiple_of(base + i * _CHUNK, 8),
                                        _CHUNK)]

            def out_slice(hop, i):
                return out.at[hop, pl.ds(pl.multiple_of(base + i * _CHUNK, 8),
                                         _CHUNK)]

            def start_idx(i, b):
                pltpu.async_copy(src_slice(i), idx[b], sidx[b])

            def wait_idx(i, b):
                pltpu.make_async_copy(src_slice(i), idx[b], sidx[b]).wait()

            def start_gather(b):
                pltpu.async_copy(table.at[idx[b]], rows[b], sg[b])

            def wait_gather(b):
                pltpu.make_async_copy(table.at[idx[b]], rows[b], sg[b]).wait()

            def start_writes(i, b):
                pltpu.async_copy(rows[b], out_slice(0, i), sw[b])
                pltpu.async_copy(rows[b], out_slice(1, i), sw[b])

            def wait_writes(i, b):
                pltpu.make_async_copy(rows[b], out_slice(0, i), sw[b]).wait()
                pltpu.make_async_copy(rows[b], out_slice(1, i), sw[b]).wait()

            for b in range(_NBUF):
                start_idx(b, b)
            for b in range(2):
                wait_idx(b, b)
                start_gather(b)

            def body(j, _):
                for u in range(_NBUF):
                    i = j * _NBUF + u
                    b = u
                    b2 = (u + 2) % _NBUF
                    wait_gather(b)
                    start_writes(i, b)
                    @pl.when(j < n_groups - 1)
                    def _():
                        start_idx(i + _NBUF, b)
                    if u < 2:
                        @pl.when(j > 0)
                        def _():
                            wait_writes(i - 2, b2)
                        wait_idx(i + 2, b2)
                        start_gather(b2)
                    else:
                        @pl.when(i + 2 < n_chunks)
                        def _():
                            wait_writes(i - 2, b2)
                            wait_idx(i + 2, b2)
                            start_gather(b2)
                return 0

            lax.fori_loop(0, n_groups, body, 0)
            for u in range(_NBUF):
                wait_writes(n_chunks - _NBUF + u, u)

        run_array(h_hbm, ent_hbm, h_out)
        run_array(r_hbm, rel_hbm, r_out)
        run_array(t_hbm, ent_hbm, t_out)

    return k


def kernel(entity_emb_matrix, relation_emb_matrix, items, memories_h,
           memories_r, memories_t):
    b, m = memories_h.shape
    n_flat = b * m
    run = _build(n_flat, b, _DIM)
    items_emb, h, r, t = run(
        entity_emb_matrix,
        relation_emb_matrix,
        items.astype(jnp.int32),
        memories_h.reshape(n_flat),
        memories_r.reshape(n_flat),
        memories_t.reshape(n_flat),
    )
    shp = (_NHOP, b, m, _DIM)
    return items_emb, h.reshape(shp), r.reshape(shp), t.reshape(shp)


# speedup vs baseline: 2.5697x; 1.8126x over previous
import functools

import jax
import jax.numpy as jnp
from jax import lax
from jax.experimental import pallas as pl
from jax.experimental.pallas import tpu as pltpu
from jax.experimental.pallas import tpu_sc as plsc

_DIM = 32
_NHOP = 2
_CHUNK = 256
_UNITS = _CHUNK // 128


@functools.lru_cache(maxsize=None)
def _build(n_batch, n_mem, dim):
    info = plsc.get_sparse_core_info()
    nw = info.num_cores * info.num_subcores
    n_flat = n_batch * n_mem
    per_w = n_flat // nw
    n_chunks = per_w // _CHUNK
    items_per_w = n_batch // nw
    assert per_w * nw == n_flat and n_chunks * _CHUNK == per_w
    assert n_chunks % 2 == 0 and items_per_w * nw == n_batch
    assert dim == 32 and n_batch % 128 == 0

    n_bblk = n_batch // 128
    mesh = plsc.VectorSubcoreMesh(core_axis_name="c", subcore_axis_name="s")
    out6 = jax.ShapeDtypeStruct(
        (_NHOP, n_mem, dim // 8, n_bblk, 8, 128), jnp.float32)

    scratch = (
        [pltpu.VMEM((_CHUNK,), jnp.int32) for _ in range(6)]
        + [pltpu.VMEM((_CHUNK, dim), jnp.float32) for _ in range(4)]
        + [pltpu.VMEM((dim // 8, _UNITS, 8, 128), jnp.float32)
           for _ in range(6)]
        + [pltpu.VMEM((32, dim), jnp.float32),
           pltpu.VMEM((items_per_w,), jnp.int32),
           pltpu.VMEM((items_per_w, dim), jnp.float32)]
        + [pltpu.SemaphoreType.DMA for _ in range(16)]
    )

    @functools.partial(
        pl.kernel,
        mesh=mesh,
        compiler_params=pltpu.CompilerParams(use_tc_tiling_on_sc=False,
                                             needs_layout_passes=False),
        out_type=[
            jax.ShapeDtypeStruct((n_batch, dim), jnp.float32),
            out6, out6, out6,
        ],
        scratch_types=scratch,
    )
    def k(ent_hbm, rel_hbm, items_hbm, h_hbm, r_hbm, t_hbm,
          items_out, h_out, r_out, t_out, *s):
        idx = {"h": s[0:2], "t": s[2:4], "r": s[4:6]}
        rows = {"h": s[6:8], "t": s[8:10]}
        trb = {"h": s[10:12], "t": s[12:14], "r": s[14:16]}
        rel_v, iidx_v, irows_v = s[16], s[17], s[18]
        sems = s[19:]
        sidx = {"h": sems[0:2], "t": sems[2:4], "r": sems[4:6]}
        sg = {"h": sems[6:8], "t": sems[8:10]}
        sw = {"h": sems[10:12], "t": sems[12:14], "r": sems[14:16]}
        src = {}
        outp = {}

        wid = lax.axis_index("s") * info.num_cores + lax.axis_index("c")
        base = wid * per_w
        unit0 = wid * (per_w // 128)

        pltpu.sync_copy(rel_hbm, rel_v.at[pl.ds(0, 26)])

        ibase = pl.multiple_of(wid * items_per_w, 8)
        pltpu.sync_copy(items_hbm.at[pl.ds(ibase, items_per_w)], iidx_v)
        pltpu.async_copy(ent_hbm.at[iidx_v], irows_v, sems[6]).wait()
        pltpu.sync_copy(irows_v, items_out.at[pl.ds(ibase, items_per_w)])

        iota16 = lax.iota(jnp.int32, 16)
        row_vecs = [iota16 + (16 * v) for v in range(_CHUNK // 16)]

        def src_slice(a, i):
            return src[a].at[pl.ds(pl.multiple_of(base + i * _CHUNK, 8),
                                   _CHUNK)]

        def start_idx(a, i, b):
            pltpu.async_copy(src_slice(a, i), idx[a][b], sidx[a][b])

        def wait_idx(a, i, b):
            pltpu.make_async_copy(src_slice(a, i), idx[a][b],
                                  sidx[a][b]).wait()

        def start_gather(a, b):
            pltpu.async_copy(ent_hbm.at[idx[a][b]], rows[a][b], sg[a][b])

        def wait_gather(a, b):
            pltpu.make_async_copy(ent_hbm.at[idx[a][b]], rows[a][b],
                                  sg[a][b]).wait()

        def unit_mbb(i):
            u = unit0 + i * _UNITS
            return u // n_bblk, u % n_bblk

        def start_writes(a, i, t):
            m, bb = unit_mbb(i)
            for hop in range(_NHOP):
                for d8 in range(dim // 8):
                    pltpu.async_copy(
                        trb[a][t].at[d8],
                        outp[a].at[hop, m, d8, pl.ds(bb, _UNITS)], sw[a][t])

        def wait_writes(a, i, t):
            m, bb = unit_mbb(i)
            for hop in range(_NHOP):
                for d8 in range(dim // 8):
                    pltpu.make_async_copy(
                        trb[a][t].at[d8],
                        outp[a].at[hop, m, d8, pl.ds(bb, _UNITS)],
                        sw[a][t]).wait()

        def transpose_rows(a, b, t):
            rows_b = rows[a][b]
            trans_t = trb[a][t]

            def body(c, _):
                d8 = c // 8
                dr = c % 8
                col = jnp.full((16,), c, jnp.int32)
                for v in range(_CHUNK // 16):
                    val = plsc.load_gather(rows_b, [row_vecs[v], col])
                    trans_t[d8, v // 8, dr, pl.ds((v % 8) * 16, 16)] = val
                return 0

            lax.fori_loop(0, dim, body, 0)

        def expand_rel(b, t):
            trans_t = trb["r"][t]
            ridx = [idx["r"][b][pl.ds(16 * v, 16)]
                    for v in range(_CHUNK // 16)]

            def body(c, _):
                d8 = c // 8
                dr = c % 8
                col = jnp.full((16,), c, jnp.int32)
                for v in range(_CHUNK // 16):
                    val = plsc.load_gather(rel_v, [ridx[v], col])
                    trans_t[d8, v // 8, dr, pl.ds((v % 8) * 16, 16)] = val
                return 0

            lax.fori_loop(0, dim, body, 0)

        def run(h_src, r_src, t_src):
            src["h"], src["r"], src["t"] = h_src, r_src, t_src
            outp["h"], outp["r"], outp["t"] = h_out, r_out, t_out

            for a in ("h", "t", "r"):
                start_idx(a, 0, 0)
                start_idx(a, 1, 1)
            for a in ("h", "t"):
                wait_idx(a, 0, 0)
                start_gather(a, 0)
                wait_idx(a, 1, 1)
                start_gather(a, 1)

            def body(j, _):
                for u in range(2):
                    i = j * 2 + u
                    for a in ("h", "t"):
                        wait_gather(a, u)
                        @pl.when(j < n_chunks // 2 - 1)
                        def _():
                            start_idx(a, i + 2, u)
                        @pl.when(j > 0)
                        def _():
                            wait_writes(a, i - 2, u)
                        transpose_rows(a, u, u)
                        @pl.when(j < n_chunks // 2 - 1)
                        def _():
                            wait_idx(a, i + 2, u)
                            start_gather(a, u)
                        start_writes(a, i, u)
                    wait_idx("r", i, u)
                    @pl.when(j > 0)
                    def _():
                        wait_writes("r", i - 2, u)
                    expand_rel(u, u)
                    start_writes("r", i, u)
                    @pl.when(j < n_chunks // 2 - 1)
                    def _():
                        start_idx("r", i + 2, u)
                return 0

            lax.fori_loop(0, n_chunks // 2, body, 0)
            for a in ("h", "t", "r"):
                wait_writes(a, n_chunks - 2, 0)
                wait_writes(a, n_chunks - 1, 1)

        run(h_hbm, r_hbm, t_hbm)

    return k


def kernel(entity_emb_matrix, relation_emb_matrix, items, memories_h,
           memories_r, memories_t):
    b, m = memories_h.shape
    run = _build(b, m, _DIM)
    items_emb, h6, r6, t6 = run(
        entity_emb_matrix,
        relation_emb_matrix,
        items.astype(jnp.int32),
        memories_h.T.reshape(-1),
        memories_r.T.reshape(-1),
        memories_t.T.reshape(-1),
    )

    def to4d(x6):
        return x6.transpose(0, 3, 5, 1, 2, 4).reshape(_NHOP, b, m, _DIM)

    return items_emb, to4d(h6), to4d(r6), to4d(t6)
